# Initial kernel scaffold; baseline (speedup 1.0000x reference)
import jax
import jax.numpy as jnp
from jax.experimental import pallas as pl


def kernel(x, edge_index, edge_weight, W1, b1, W2, b2):
    raise NotImplementedError("write your pallas kernel here")


# speedup vs baseline: 2.7667x; 2.7667x over previous
import functools

import jax
import jax.numpy as jnp
from jax import lax
from jax.experimental import pallas as pl
from jax.experimental.pallas import tpu as pltpu
from jax.experimental.pallas import tpu_sc as plsc

N_NODES = 10000
N_PAD = 10240
N_FEATS = 16
F_PAD = 128
LANES = 16
CHUNK = 128
NW = 32


def _sc_segment_sum(h, src1d, dst1d, wrep, zeros, chunks_per_tile):
    mesh = plsc.VectorSubcoreMesh(core_axis_name="c", subcore_axis_name="s")

    @functools.partial(
        pl.kernel,
        mesh=mesh,
        out_type=jax.ShapeDtypeStruct((2, N_PAD, F_PAD), jnp.float32),
        scratch_types=[
            pltpu.VMEM((CHUNK,), jnp.int32),
            pltpu.VMEM((CHUNK,), jnp.int32),
            pltpu.VMEM((CHUNK, LANES), jnp.float32),
            pltpu.VMEM((CHUNK, F_PAD), jnp.float32),
            pltpu.VMEM_SHARED((N_PAD, F_PAD), jnp.float32),
            pltpu.SemaphoreType.DMA,
        ],
    )
    def seg_sum(h_hbm, src_hbm, dst_hbm, w_hbm, z_hbm, out_hbm,
                src_c, dst_c, w_c, rows_v, acc, sem):
        cid = lax.axis_index("c")
        sid = lax.axis_index("s")
        wid = sid * 2 + cid
        base = wid * chunks_per_tile

        @pl.when(sid == 0)
        def _zero_acc():
            pltpu.sync_copy(z_hbm, acc)

        plsc.subcore_barrier()

        def chunk_body(i, carry):
            eb = pl.ds((base + i) * CHUNK, CHUNK)
            pltpu.sync_copy(src_hbm.at[eb], src_c)
            pltpu.sync_copy(dst_hbm.at[eb], dst_c)
            pltpu.sync_copy(w_hbm.at[eb], w_c)
            pltpu.async_copy(h_hbm.at[src_c], rows_v, sem).wait()
            for e in range(CHUNK):
                rows_v[e, pl.ds(0, LANES)] = (
                    rows_v[e, pl.ds(0, LANES)] * w_c[e, pl.ds(0, LANES)])
            pltpu.sync_copy(rows_v, acc.at[dst_c], add=True)
            return carry

        lax.fori_loop(0, chunks_per_tile, chunk_body, 0)
        plsc.subcore_barrier()

        @pl.when(sid == 0)
        def _dump():
            pltpu.sync_copy(acc, out_hbm.at[cid])

    return seg_sum(h, src1d, dst1d, wrep, zeros)


def _tc_matmul1(x, W1p):
    def body(x_ref, w_ref, o_ref):
        o_ref[...] = jnp.dot(x_ref[...], w_ref[...],
                             preferred_element_type=jnp.float32,
                             precision=lax.Precision.HIGHEST)

    return pl.pallas_call(
        body,
        grid=(10,),
        in_specs=[
            pl.BlockSpec((N_PAD // 10, 128), lambda i: (i, 0)),
            pl.BlockSpec((128, F_PAD), lambda i: (0, 0)),
        ],
        out_specs=pl.BlockSpec((N_PAD // 10, F_PAD), lambda i: (i, 0)),
        out_shape=jax.ShapeDtypeStruct((N_PAD, F_PAD), jnp.float32),
    )(x, W1p)


def _tc_mid(p0, p1, b1p, W2p):
    def body(p0_ref, p1_ref, b_ref, w_ref, o_ref):
        hidden = jnp.maximum(p0_ref[...] + p1_ref[...] + b_ref[...], 0.0)
        o_ref[...] = jnp.dot(hidden, w_ref[...],
                             preferred_element_type=jnp.float32,
                             precision=lax.Precision.HIGHEST)

    return pl.pallas_call(
        body,
        grid=(10,),
        in_specs=[
            pl.BlockSpec((N_PAD // 10, F_PAD), lambda i: (i, 0)),
            pl.BlockSpec((N_PAD // 10, F_PAD), lambda i: (i, 0)),
            pl.BlockSpec((1, F_PAD), lambda i: (0, 0)),
            pl.BlockSpec((F_PAD, F_PAD), lambda i: (0, 0)),
        ],
        out_specs=pl.BlockSpec((N_PAD // 10, F_PAD), lambda i: (i, 0)),
        out_shape=jax.ShapeDtypeStruct((N_PAD, F_PAD), jnp.float32),
    )(p0, p1, b1p, W2p)


def _tc_final(p0, p1, b2):
    def body(p0_ref, p1_ref, b_ref, o_ref):
        z = p0_ref[...] + p1_ref[...] + b_ref[...]
        m = jnp.max(z, axis=1, keepdims=True)
        lse = jnp.log(jnp.sum(jnp.exp(z - m), axis=1, keepdims=True))
        o_ref[...] = z - m - lse

    return pl.pallas_call(
        body,
        grid=(10,),
        in_specs=[
            pl.BlockSpec((N_NODES // 10, N_FEATS), lambda i: (i, 0)),
            pl.BlockSpec((N_NODES // 10, N_FEATS), lambda i: (i, 0)),
            pl.BlockSpec((1, N_FEATS), lambda i: (0, 0)),
        ],
        out_specs=pl.BlockSpec((N_NODES // 10, N_FEATS), lambda i: (i, 0)),
        out_shape=jax.ShapeDtypeStruct((N_NODES, N_FEATS), jnp.float32),
    )(p0, p1, b2)


def kernel(x, edge_index, edge_weight, W1, b1, W2, b2):
    E = edge_index.shape[1]
    n_chunks = -(-E // CHUNK)
    chunks_per_tile = -(-n_chunks // NW)
    EP = NW * chunks_per_tile * CHUNK

    src = edge_index[0].astype(jnp.int32)
    dst = edge_index[1].astype(jnp.int32)
    w = edge_weight.astype(jnp.float32)
    pad = EP - E
    src1d = jnp.concatenate([src, jnp.zeros((pad,), jnp.int32)])
    dst1d = jnp.concatenate([dst, jnp.zeros((pad,), jnp.int32)])
    w1d = jnp.concatenate([w, jnp.zeros((pad,), jnp.float32)])
    wrep = jnp.broadcast_to(w1d[:, None], (EP, LANES))
    zeros = jnp.zeros((N_PAD, F_PAD), jnp.float32)

    x_pad = jnp.pad(x, ((0, N_PAD - N_NODES), (0, 0)))
    W1p = jnp.pad(W1, ((0, 0), (0, F_PAD - N_FEATS)))
    W2p = jnp.pad(W2, ((0, F_PAD - N_FEATS), (0, F_PAD - N_FEATS)))
    b1p = jnp.pad(b1, (0, F_PAD - N_FEATS)).reshape(1, F_PAD)

    h1 = _tc_matmul1(x_pad, W1p)
    p1 = _sc_segment_sum(h1, src1d, dst1d, wrep, zeros, chunks_per_tile)
    h2 = _tc_mid(p1[0], p1[1], b1p, W2p)
    p2 = _sc_segment_sum(h2, src1d, dst1d, wrep, zeros, chunks_per_tile)
    return _tc_final(p2[0, :N_NODES, :N_FEATS], p2[1, :N_NODES, :N_FEATS],
                     b2.reshape(1, N_FEATS))


# speedup vs baseline: 2.8444x; 1.0281x over previous
import functools

import jax
import jax.numpy as jnp
from jax import lax
from jax.experimental import pallas as pl
from jax.experimental.pallas import tpu as pltpu
from jax.experimental.pallas import tpu_sc as plsc

N_NODES = 10000
N_PAD = 10240
N_FEATS = 16
F_PAD = 128
LANES = 16
CHUNK = 128
NW = 32


def _sc_segment_sum(h, src3d, dst3d, wrep, zeros, chunks_per_tile):
    mesh = plsc.VectorSubcoreMesh(core_axis_name="c", subcore_axis_name="s")
    cpt = chunks_per_tile

    @functools.partial(
        pl.kernel,
        mesh=mesh,
        out_type=jax.ShapeDtypeStruct((2, N_PAD, F_PAD), jnp.float32),
        scratch_types=[
            pltpu.VMEM((CHUNK,), jnp.int32),
            pltpu.VMEM((cpt, CHUNK), jnp.int32),
            pltpu.VMEM((CHUNK, LANES), jnp.float32),
            pltpu.VMEM((CHUNK, F_PAD), jnp.float32),
            pltpu.VMEM_SHARED((N_PAD, F_PAD), jnp.float32),
            pltpu.SemaphoreType.DMA,
        ],
    )
    def seg_sum(h_hbm, src_hbm, dst_hbm, w_hbm, z_hbm, out_hbm,
                src_c, dst_t, w_c, rows_v, acc, sem):
        cid = lax.axis_index("c")
        sid = lax.axis_index("s")
        wid = sid * 2 + cid
        base = wid * cpt

        @pl.when(sid == 0)
        def _zero_acc():
            pltpu.sync_copy(z_hbm, acc)

        pltpu.sync_copy(dst_hbm.at[wid], dst_t)
        plsc.subcore_barrier()

        def chunk_body(i, carry):
            eb = pl.ds((base + i) * CHUNK, CHUNK)
            pltpu.sync_copy(src_hbm.at[wid, i], src_c)
            pltpu.sync_copy(w_hbm.at[eb], w_c)
            pltpu.async_copy(h_hbm.at[src_c], rows_v, sem).wait()
            for e in range(CHUNK):
                rows_v[e, pl.ds(0, LANES)] = (
                    rows_v[e, pl.ds(0, LANES)] * w_c[e, pl.ds(0, LANES)])
            pltpu.sync_copy(rows_v, acc.at[dst_t.at[i]], add=True)
            return carry

        lax.fori_loop(0, cpt, chunk_body, 0)
        plsc.subcore_barrier()

        @pl.when(sid == 0)
        def _dump():
            pltpu.sync_copy(acc, out_hbm.at[cid])

    return seg_sum(h, src3d, dst3d, wrep, zeros)


def _tc_matmul1(x, W1p):
    def body(x_ref, w_ref, o_ref):
        o_ref[...] = jnp.dot(x_ref[...], w_ref[...],
                             preferred_element_type=jnp.float32,
                             precision=lax.Precision.HIGHEST)

    return pl.pallas_call(
        body,
        grid=(10,),
        in_specs=[
            pl.BlockSpec((N_PAD // 10, 128), lambda i: (i, 0)),
            pl.BlockSpec((128, F_PAD), lambda i: (0, 0)),
        ],
        out_specs=pl.BlockSpec((N_PAD // 10, F_PAD), lambda i: (i, 0)),
        out_shape=jax.ShapeDtypeStruct((N_PAD, F_PAD), jnp.float32),
    )(x, W1p)


def _tc_mid(p0, p1, b1p, W2p):
    def body(p0_ref, p1_ref, b_ref, w_ref, o_ref):
        hidden = jnp.maximum(p0_ref[...] + p1_ref[...] + b_ref[...], 0.0)
        o_ref[...] = jnp.dot(hidden, w_ref[...],
                             preferred_element_type=jnp.float32,
                             precision=lax.Precision.HIGHEST)

    return pl.pallas_call(
        body,
        grid=(10,),
        in_specs=[
            pl.BlockSpec((N_PAD // 10, F_PAD), lambda i: (i, 0)),
            pl.BlockSpec((N_PAD // 10, F_PAD), lambda i: (i, 0)),
            pl.BlockSpec((1, F_PAD), lambda i: (0, 0)),
            pl.BlockSpec((F_PAD, F_PAD), lambda i: (0, 0)),
        ],
        out_specs=pl.BlockSpec((N_PAD // 10, F_PAD), lambda i: (i, 0)),
        out_shape=jax.ShapeDtypeStruct((N_PAD, F_PAD), jnp.float32),
    )(p0, p1, b1p, W2p)


def _tc_final(p0, p1, b2):
    def body(p0_ref, p1_ref, b_ref, o_ref):
        z = p0_ref[...] + p1_ref[...] + b_ref[...]
        m = jnp.max(z, axis=1, keepdims=True)
        lse = jnp.log(jnp.sum(jnp.exp(z - m), axis=1, keepdims=True))
        o_ref[...] = z - m - lse

    return pl.pallas_call(
        body,
        grid=(10,),
        in_specs=[
            pl.BlockSpec((N_NODES // 10, N_FEATS), lambda i: (i, 0)),
            pl.BlockSpec((N_NODES // 10, N_FEATS), lambda i: (i, 0)),
            pl.BlockSpec((1, N_FEATS), lambda i: (0, 0)),
        ],
        out_specs=pl.BlockSpec((N_NODES // 10, N_FEATS), lambda i: (i, 0)),
        out_shape=jax.ShapeDtypeStruct((N_NODES, N_FEATS), jnp.float32),
    )(p0, p1, b2)


def kernel(x, edge_index, edge_weight, W1, b1, W2, b2):
    E = edge_index.shape[1]
    n_chunks = -(-E // CHUNK)
    chunks_per_tile = -(-n_chunks // NW)
    EP = NW * chunks_per_tile * CHUNK

    src = edge_index[0].astype(jnp.int32)
    dst = edge_index[1].astype(jnp.int32)
    w = edge_weight.astype(jnp.float32)
    pad = EP - E
    src3d = jnp.concatenate([src, jnp.zeros((pad,), jnp.int32)]).reshape(
        NW, chunks_per_tile, CHUNK)
    dst3d = jnp.concatenate([dst, jnp.zeros((pad,), jnp.int32)]).reshape(
        NW, chunks_per_tile, CHUNK)
    w1d = jnp.concatenate([w, jnp.zeros((pad,), jnp.float32)])
    wrep = jnp.broadcast_to(w1d[:, None], (EP, LANES))
    zeros = jnp.zeros((N_PAD, F_PAD), jnp.float32)

    x_pad = jnp.pad(x, ((0, N_PAD - N_NODES), (0, 0)))
    W1p = jnp.pad(W1, ((0, 0), (0, F_PAD - N_FEATS)))
    W2p = jnp.pad(W2, ((0, F_PAD - N_FEATS), (0, F_PAD - N_FEATS)))
    b1p = jnp.pad(b1, (0, F_PAD - N_FEATS)).reshape(1, F_PAD)

    h1 = _tc_matmul1(x_pad, W1p)
    p1 = _sc_segment_sum(h1, src3d, dst3d, wrep, zeros, chunks_per_tile)
    h2 = _tc_mid(p1[0], p1[1], b1p, W2p)
    p2 = _sc_segment_sum(h2, src3d, dst3d, wrep, zeros, chunks_per_tile)
    return _tc_final(p2[0, :N_NODES, :N_FEATS], p2[1, :N_NODES, :N_FEATS],
                     b2.reshape(1, N_FEATS))
